# baseline (speedup 1.0000x reference)
import functools

import jax
import jax.numpy as jnp
from jax import lax
from jax.experimental import pallas as pl
from jax.experimental.pallas import tpu as pltpu
from jax.experimental.pallas import tpu_sc as plsc

_ROWS = 16906
_DIM = 200
_TOTAL = _ROWS * _DIM
_TABLE_TOTAL = (_ROWS + 1) * _DIM
_NW = 32
_CHUNK = 105664
_LAST_BASE = _TOTAL - _CHUNK

_mesh = plsc.VectorSubcoreMesh(core_axis_name="c", subcore_axis_name="s")


@functools.partial(
    pl.kernel,
    mesh=_mesh,
    out_type=jax.ShapeDtypeStruct((_TOTAL,), jnp.float32),
    scratch_types=[pltpu.VMEM((_CHUNK,), jnp.float32)],
)
def _copy_kernel(tab_hbm, out_hbm, buf):
    wid = lax.axis_index("s") * 2 + lax.axis_index("c")
    base = jnp.minimum(wid * _CHUNK, _LAST_BASE)
    pltpu.sync_copy(tab_hbm.at[pl.ds(base, _CHUNK)], buf)
    pltpu.sync_copy(buf, out_hbm.at[pl.ds(base, _CHUNK)])


def kernel(x, table):
    del x
    flat = table.reshape(_TABLE_TOTAL)
    return _copy_kernel(flat).reshape(_ROWS, _DIM)


# speedup vs baseline: 3.8331x; 3.8331x over previous
import functools

import jax
import jax.numpy as jnp
from jax import lax
from jax.experimental import pallas as pl
from jax.experimental.pallas import tpu as pltpu
from jax.experimental.pallas import tpu_sc as plsc

_ROWS = 16906
_DIM = 200
_NW = 32
_CHUNK = 528
_TAIL_BLOCK = 16

_mesh = plsc.VectorSubcoreMesh(core_axis_name="c", subcore_axis_name="s")


@functools.partial(
    pl.kernel,
    mesh=_mesh,
    out_type=jax.ShapeDtypeStruct((_ROWS, _DIM), jnp.float32),
    scratch_types=[pltpu.VMEM((_CHUNK // 2, _DIM), jnp.float32)],
)
def _sc_bulk_copy(tab_hbm, out_hbm, buf):
    wid = lax.axis_index("s") * 2 + lax.axis_index("c")
    half = _CHUNK // 2
    for p in range(2):
        base = wid * _CHUNK + p * half
        pltpu.sync_copy(tab_hbm.at[pl.ds(base, half)], buf)
        pltpu.sync_copy(buf, out_hbm.at[pl.ds(base, half)])


def _tc_tail_body(acc_ref, tab_ref, out_ref):
    del acc_ref
    out_ref[...] = tab_ref[...]


_tc_tail = pl.pallas_call(
    _tc_tail_body,
    grid=(1,),
    in_specs=[
        pl.BlockSpec(memory_space=pl.ANY),
        pl.BlockSpec((_TAIL_BLOCK, _DIM), lambda i: (_ROWS // _TAIL_BLOCK, 0)),
    ],
    out_specs=pl.BlockSpec((_TAIL_BLOCK, _DIM),
                           lambda i: (_ROWS // _TAIL_BLOCK, 0)),
    out_shape=jax.ShapeDtypeStruct((_ROWS, _DIM), jnp.float32),
    input_output_aliases={0: 0},
)


def kernel(x, table):
    del x
    bulk = _sc_bulk_copy(table)
    return _tc_tail(bulk, table)


# speedup vs baseline: 7.4673x; 1.9481x over previous
import functools

import jax
import jax.numpy as jnp
from jax import lax
from jax.experimental import pallas as pl
from jax.experimental.pallas import tpu as pltpu
from jax.experimental.pallas import tpu_sc as plsc

_ROWS = 16906
_TAB = 16907
_DIM = 200
_NW = 32
_CHUNK = 512
_BULK = _NW * _CHUNK
_TC_BLKS = 5
_TC_PAD = _TC_BLKS * 128

_mesh = plsc.VectorSubcoreMesh(core_axis_name="c", subcore_axis_name="s")


@functools.partial(
    pl.kernel,
    mesh=_mesh,
    out_type=jax.ShapeDtypeStruct((_DIM, _ROWS), jnp.float32),
    scratch_types=[pltpu.VMEM((_DIM, _CHUNK), jnp.float32)],
)
def _sc_bulk_copy(tab_hbm, out_hbm, buf):
    wid = lax.axis_index("s") * 2 + lax.axis_index("c")
    base = wid * _CHUNK
    pltpu.sync_copy(tab_hbm.at[:, pl.ds(base, _CHUNK)], buf)
    pltpu.sync_copy(buf, out_hbm.at[:, pl.ds(base, _CHUNK)])


def _tc_tail_body(acc_ref, tab_ref, out_ref):
    del acc_ref
    out_ref[...] = tab_ref[...]


_tc_tail = pl.pallas_call(
    _tc_tail_body,
    grid=(_TC_BLKS,),
    in_specs=[
        pl.BlockSpec(memory_space=pltpu.MemorySpace.HBM),
        pl.BlockSpec((_DIM, 128), lambda j: (0, j)),
    ],
    out_specs=pl.BlockSpec((_DIM, 128), lambda j: (0, _BULK // 128 + j)),
    out_shape=jax.ShapeDtypeStruct((_DIM, _ROWS), jnp.float32),
    input_output_aliases={0: 0},
)


def kernel(x, table):
    del x
    tab_t = table.T
    bulk = _sc_bulk_copy(tab_t)
    tail = lax.slice(tab_t, (0, _BULK), (_DIM, _TAB))
    tail = jnp.pad(tail, ((0, 0), (0, _TC_PAD - (_TAB - _BULK))))
    out_t = _tc_tail(bulk, tail)
    return out_t.T


# speedup vs baseline: 7.7993x; 1.0445x over previous
import functools

import jax
import jax.numpy as jnp
from jax import lax
from jax.experimental import pallas as pl
from jax.experimental.pallas import tpu as pltpu
from jax.experimental.pallas import tpu_sc as plsc

_ROWS = 16906
_TAB = 16907
_DIM = 200
_NW = 32
_CHUNK = 512
_BULK = _NW * _CHUNK
_TC_BLK = 1024
_TC_PAD = _TC_BLK

_mesh = plsc.VectorSubcoreMesh(core_axis_name="c", subcore_axis_name="s")


@functools.partial(
    pl.kernel,
    mesh=_mesh,
    out_type=jax.ShapeDtypeStruct((_DIM, _ROWS), jnp.float32),
    scratch_types=[pltpu.VMEM((_DIM, _CHUNK), jnp.float32)],
)
def _sc_bulk_copy(tab_hbm, out_hbm, buf):
    wid = lax.axis_index("s") * 2 + lax.axis_index("c")
    base = wid * _CHUNK
    pltpu.sync_copy(tab_hbm.at[:, pl.ds(base, _CHUNK)], buf)
    pltpu.sync_copy(buf, out_hbm.at[:, pl.ds(base, _CHUNK)])


def _tc_tail_body(acc_ref, tab_ref, out_ref):
    del acc_ref
    out_ref[...] = tab_ref[...]


_tc_tail = pl.pallas_call(
    _tc_tail_body,
    grid=(1,),
    in_specs=[
        pl.BlockSpec(memory_space=pltpu.MemorySpace.HBM),
        pl.BlockSpec((_DIM, _TC_BLK), lambda j: (0, 0)),
    ],
    out_specs=pl.BlockSpec((_DIM, _TC_BLK), lambda j: (0, _BULK // _TC_BLK)),
    out_shape=jax.ShapeDtypeStruct((_DIM, _ROWS), jnp.float32),
    input_output_aliases={0: 0},
)


def kernel(x, table):
    del x
    tab_t = table.T
    bulk = _sc_bulk_copy(tab_t)
    tail = lax.slice(tab_t, (0, _BULK), (_DIM, _TAB))
    tail = jnp.pad(tail, ((0, 0), (0, _TC_PAD - (_TAB - _BULK))))
    out_t = _tc_tail(bulk, tail)
    return out_t.T
